# Initial kernel scaffold; baseline (speedup 1.0000x reference)
import jax
import jax.numpy as jnp
from jax.experimental import pallas as pl


def kernel(x, pos, batch, W1, b1, W2, b2, Wg, bg):
    raise NotImplementedError("write your pallas kernel here")


# speedup vs baseline: 3.3009x; 3.3009x over previous
import jax
import jax.numpy as jnp
from jax.experimental import pallas as pl
from jax.experimental.pallas import tpu as pltpu

N = 8192
K = 16
BQ = 256
NBLK = N // BQ


def _knn_mlp_kernel(pos_q_r, x_q_r, bat_q_r, posT_r, batT_r, pos_f_r,
                    W1x_r, W1p_r, b1_r, W2_r, b2_r,
                    nbr_ref, h2_ref, h2s_ref):
    pos_q, x_q, bat_q = pos_q_r[...], x_q_r[...], bat_q_r[...]
    posT, batT, pos_f = posT_r[...], batT_r[...], pos_f_r[...]
    W1x, W1p, b1 = W1x_r[...], W1p_r[...], b1_r[...]
    W2, b2 = W2_r[...], b2_r[...]
    d0 = pos_q[:, 0:1] - posT[0:1, :]
    d1 = pos_q[:, 1:2] - posT[1:2, :]
    d2c = pos_q[:, 2:3] - posT[2:3, :]
    d = (d0 * d0 + d1 * d1) + d2c * d2c
    d = jnp.where(bat_q != batT, jnp.inf, d)

    base = jnp.dot(x_q, W1x, preferred_element_type=jnp.float32) + b1
    h1s = jax.nn.relu(base)
    h2s_ref[...] = jax.nn.relu(
        jnp.dot(h1s, W2, preferred_element_type=jnp.float32) + b2)

    ii = jax.lax.broadcasted_iota(jnp.int32, (BQ, N), 1)
    for k in range(K):
        m = jnp.min(d, axis=1, keepdims=True)
        masked = jnp.where(d == m, ii, jnp.int32(N))
        amin = jnp.min(masked, axis=1)
        onehot = ii == amin[:, None]
        nbr_ref[k, :] = amin
        pos_n = jnp.dot(onehot.astype(jnp.float32), pos_f,
                        preferred_element_type=jnp.float32)
        d = jnp.where(onehot, jnp.inf, d)
        pd = pos_q - pos_n
        h1 = jax.nn.relu(base + jnp.dot(pd, W1p,
                                        preferred_element_type=jnp.float32))
        h2_ref[k] = jax.nn.relu(
            jnp.dot(h1, W2, preferred_element_type=jnp.float32) + b2)


def _scatter_kernel(nbr_sm, h2_ref, h2s_ref, out_ref):
    i = pl.program_id(0)

    @pl.when(i == 0)
    def _init():
        out_ref[...] = h2s_ref[...]

    def body(q, carry):
        for k in range(K):
            dst = nbr_sm[k, i * BQ + q]
            row = h2_ref[k, pl.ds(q, 1), :]
            cur = out_ref[pl.ds(dst, 1), :]
            out_ref[pl.ds(dst, 1), :] = jnp.maximum(cur, row)
        return carry

    jax.lax.fori_loop(0, BQ, body, 0)


def _final_kernel(agg, Wg, bg, out_ref):
    out_ref[...] = jax.nn.relu(
        jnp.dot(agg[...], Wg[...], preferred_element_type=jnp.float32) + bg[...])


def kernel(x, pos, batch, W1, b1, W2, b2, Wg, bg):
    batf = batch.astype(jnp.float32)
    posT = pos.T
    W1x = W1[:128, :]
    W1p = W1[128:, :]

    nbr, h2, h2s = pl.pallas_call(
        _knn_mlp_kernel,
        grid=(NBLK,),
        in_specs=[
            pl.BlockSpec((BQ, 3), lambda i: (i, 0)),
            pl.BlockSpec((BQ, 128), lambda i: (i, 0)),
            pl.BlockSpec((BQ, 1), lambda i: (i, 0)),
            pl.BlockSpec((3, N), lambda i: (0, 0)),
            pl.BlockSpec((1, N), lambda i: (0, 0)),
            pl.BlockSpec((N, 3), lambda i: (0, 0)),
            pl.BlockSpec((128, 128), lambda i: (0, 0)),
            pl.BlockSpec((3, 128), lambda i: (0, 0)),
            pl.BlockSpec((1, 128), lambda i: (0, 0)),
            pl.BlockSpec((128, 256), lambda i: (0, 0)),
            pl.BlockSpec((1, 256), lambda i: (0, 0)),
        ],
        out_specs=[
            pl.BlockSpec((K, BQ), lambda i: (0, i)),
            pl.BlockSpec((K, BQ, 256), lambda i: (0, i, 0)),
            pl.BlockSpec((BQ, 256), lambda i: (i, 0)),
        ],
        out_shape=[
            jax.ShapeDtypeStruct((K, N), jnp.int32),
            jax.ShapeDtypeStruct((K, N, 256), jnp.float32),
            jax.ShapeDtypeStruct((N, 256), jnp.float32),
        ],
    )(pos, x, batf.reshape(N, 1), posT, batf.reshape(1, N), pos,
      W1x, W1p, b1.reshape(1, 128), W2, b2.reshape(1, 256))

    agg = pl.pallas_call(
        _scatter_kernel,
        grid_spec=pltpu.PrefetchScalarGridSpec(
            num_scalar_prefetch=1,
            grid=(NBLK,),
            in_specs=[
                pl.BlockSpec((K, BQ, 256), lambda i, s: (0, i, 0)),
                pl.BlockSpec((N, 256), lambda i, s: (0, 0)),
            ],
            out_specs=pl.BlockSpec((N, 256), lambda i, s: (0, 0)),
        ),
        out_shape=jax.ShapeDtypeStruct((N, 256), jnp.float32),
    )(nbr, h2, h2s)

    out = pl.pallas_call(
        _final_kernel,
        grid=(NBLK,),
        in_specs=[
            pl.BlockSpec((BQ, 256), lambda i: (i, 0)),
            pl.BlockSpec((256, 256), lambda i: (0, 0)),
            pl.BlockSpec((1, 256), lambda i: (0, 0)),
        ],
        out_specs=pl.BlockSpec((BQ, 256), lambda i: (i, 0)),
        out_shape=jax.ShapeDtypeStruct((N, 256), jnp.float32),
    )(agg, Wg, bg.reshape(1, 256))

    return (out, pos, batch)


# speedup vs baseline: 5.7920x; 1.7547x over previous
import jax
import jax.numpy as jnp
from jax.experimental import pallas as pl
from jax.experimental.pallas import tpu as pltpu

N = 8192
K = 16
BQ = 256
NBLK = N // BQ
CG = 1024
NCH = N // CG


def _make_knn_body(nch):
    width = nch * CG

    def body(ws_ref, *refs):
        (pos_q_r, x_q_r, bat_q_r), rest = refs[:3], refs[3:]
        posT_rs, rest = rest[:nch], rest[nch:]
        batT_rs, rest = rest[:nch], rest[nch:]
        posf_rs, rest = rest[:nch], rest[nch:]
        W1x_r, W1p_r, b1_r, W2_r, b2_r, nbr_ref, h2_ref, h2s_ref = rest

        i = pl.program_id(0)
        col0 = ws_ref[i] * CG
        pos_q, x_q, bat_q = pos_q_r[...], x_q_r[...], bat_q_r[...]
        posT = jnp.concatenate([r[...] for r in posT_rs], axis=1)
        batT = jnp.concatenate([r[...] for r in batT_rs], axis=1)
        pos_f = jnp.concatenate([r[...] for r in posf_rs], axis=0)
        W1x, W1p, b1 = W1x_r[...], W1p_r[...], b1_r[...]
        W2, b2 = W2_r[...], b2_r[...]

        ii = jax.lax.broadcasted_iota(jnp.int32, (BQ, width), 1)
        d0 = pos_q[:, 0:1] - posT[0:1, :]
        d1 = pos_q[:, 1:2] - posT[1:2, :]
        d2c = pos_q[:, 2:3] - posT[2:3, :]
        d = (d0 * d0 + d1 * d1) + d2c * d2c
        d = jnp.where((bat_q != batT) | (ii + col0 >= N), jnp.inf, d)

        base = jnp.dot(x_q, W1x, preferred_element_type=jnp.float32) + b1
        h1s = jax.nn.relu(base)
        h2s_ref[...] = jax.nn.relu(
            jnp.dot(h1s, W2, preferred_element_type=jnp.float32) + b2)

        for k in range(K):
            m = jnp.min(d, axis=1, keepdims=True)
            masked = jnp.where(d == m, ii, jnp.int32(width))
            amin = jnp.min(masked, axis=1)
            onehot = ii == amin[:, None]
            nbr_ref[k, :] = amin + col0
            pos_n = jnp.dot(onehot.astype(jnp.float32), pos_f,
                            preferred_element_type=jnp.float32)
            d = jnp.where(onehot, jnp.inf, d)
            pd = pos_q - pos_n
            h1 = jax.nn.relu(
                base + jnp.dot(pd, W1p, preferred_element_type=jnp.float32))
            h2_ref[k] = jax.nn.relu(
                jnp.dot(h1, W2, preferred_element_type=jnp.float32) + b2)

    return body


def _chunk_map(c):
    return lambda i, ws: (0, jnp.minimum(ws[i] + c, NCH - 1))


def _chunk_map_r(c):
    return lambda i, ws: (jnp.minimum(ws[i] + c, NCH - 1), 0)


def _knn_call(nch, ws, pos, x, bat_col, posT, bat_row, W1x, W1p, b1, W2, b2):
    in_specs = [
        pl.BlockSpec((BQ, 3), lambda i, ws: (i, 0)),
        pl.BlockSpec((BQ, 128), lambda i, ws: (i, 0)),
        pl.BlockSpec((BQ, 1), lambda i, ws: (i, 0)),
    ]
    in_specs += [pl.BlockSpec((3, CG), _chunk_map(c)) for c in range(nch)]
    in_specs += [pl.BlockSpec((1, CG), _chunk_map(c)) for c in range(nch)]
    in_specs += [pl.BlockSpec((CG, 3), _chunk_map_r(c)) for c in range(nch)]
    in_specs += [
        pl.BlockSpec((128, 128), lambda i, ws: (0, 0)),
        pl.BlockSpec((3, 128), lambda i, ws: (0, 0)),
        pl.BlockSpec((1, 128), lambda i, ws: (0, 0)),
        pl.BlockSpec((128, 256), lambda i, ws: (0, 0)),
        pl.BlockSpec((1, 256), lambda i, ws: (0, 0)),
    ]
    args = ([pos, x, bat_col] + [posT] * nch + [bat_row] * nch + [pos] * nch
            + [W1x, W1p, b1, W2, b2])
    return pl.pallas_call(
        _make_knn_body(nch),
        grid_spec=pltpu.PrefetchScalarGridSpec(
            num_scalar_prefetch=1,
            grid=(NBLK,),
            in_specs=in_specs,
            out_specs=[
                pl.BlockSpec((K, BQ), lambda i, ws: (0, i)),
                pl.BlockSpec((K, BQ, 256), lambda i, ws: (0, i, 0)),
                pl.BlockSpec((BQ, 256), lambda i, ws: (i, 0)),
            ],
        ),
        out_shape=[
            jax.ShapeDtypeStruct((K, N), jnp.int32),
            jax.ShapeDtypeStruct((K, N, 256), jnp.float32),
            jax.ShapeDtypeStruct((N, 256), jnp.float32),
        ],
    )(ws, *args)


def _scatter_kernel(nbr_sm, h2_ref, h2s_ref, out_ref):
    i = pl.program_id(0)

    @pl.when(i == 0)
    def _init():
        out_ref[...] = h2s_ref[...]

    def body(q, carry):
        for k in range(K):
            dst = nbr_sm[k, i * BQ + q]
            row = h2_ref[k, pl.ds(q, 1), :]
            cur = out_ref[pl.ds(dst, 1), :]
            out_ref[pl.ds(dst, 1), :] = jnp.maximum(cur, row)
        return carry

    jax.lax.fori_loop(0, BQ, body, 0)


def _final_kernel(agg, Wg, bg, out_ref):
    out_ref[...] = jax.nn.relu(
        jnp.dot(agg[...], Wg[...], preferred_element_type=jnp.float32) + bg[...])


def kernel(x, pos, batch, W1, b1, W2, b2, Wg, bg):
    bat32 = batch.astype(jnp.int32)
    batf = bat32.astype(jnp.float32)
    posT = pos.T
    W1x = W1[:128, :]
    W1p = W1[128:, :]
    b1r = b1.reshape(1, 128)
    b2r = b2.reshape(1, 256)
    bat_col = batf.reshape(N, 1)
    bat_row = batf.reshape(1, N)

    counts = jnp.bincount(bat32, length=8)
    off = jnp.cumsum(counts) - counts
    ends = off + counts
    qb = jnp.arange(NBLK)
    b_first = bat32[qb * BQ]
    b_last = bat32[qb * BQ + BQ - 1]
    ws = (off[b_first] // CG).astype(jnp.int32)
    fits = jnp.all(ends[b_last] - ws * CG <= 4 * CG)
    ws_full = jnp.zeros((NBLK,), jnp.int32)

    common = (pos, x, bat_col, posT, bat_row, W1x, W1p, b1r, W2, b2r)
    nbr, h2, h2s = jax.lax.cond(
        fits,
        lambda ws_, ws_full_, *a: _knn_call(4, ws_, *a),
        lambda ws_, ws_full_, *a: _knn_call(NCH, ws_full_, *a),
        ws, ws_full, *common)

    agg = pl.pallas_call(
        _scatter_kernel,
        grid_spec=pltpu.PrefetchScalarGridSpec(
            num_scalar_prefetch=1,
            grid=(NBLK,),
            in_specs=[
                pl.BlockSpec((K, BQ, 256), lambda i, s: (0, i, 0)),
                pl.BlockSpec((N, 256), lambda i, s: (0, 0)),
            ],
            out_specs=pl.BlockSpec((N, 256), lambda i, s: (0, 0)),
        ),
        out_shape=jax.ShapeDtypeStruct((N, 256), jnp.float32),
    )(nbr, h2, h2s)

    out = pl.pallas_call(
        _final_kernel,
        grid=(NBLK,),
        in_specs=[
            pl.BlockSpec((BQ, 256), lambda i: (i, 0)),
            pl.BlockSpec((256, 256), lambda i: (0, 0)),
            pl.BlockSpec((1, 256), lambda i: (0, 0)),
        ],
        out_specs=pl.BlockSpec((BQ, 256), lambda i: (i, 0)),
        out_shape=jax.ShapeDtypeStruct((N, 256), jnp.float32),
    )(agg, Wg, bg.reshape(1, 256))

    return (out, pos, batch)


# speedup vs baseline: 6.8628x; 1.1849x over previous
import jax
import jax.numpy as jnp
from jax.experimental import pallas as pl
from jax.experimental.pallas import tpu as pltpu

N = 8192
K = 16
BQ = 256
NBLK = N // BQ
CG = 512
NCH = N // CG


def _make_knn_body(nch):
    width = nch * CG

    def body(ws_ref, *refs):
        (pos_q_r, x_q_r, bat_q_r), rest = refs[:3], refs[3:]
        posT_rs, rest = rest[:nch], rest[nch:]
        batT_rs, rest = rest[:nch], rest[nch:]
        posf_rs, rest = rest[:nch], rest[nch:]
        W1x_r, W1p_r, b1_r, W2_r, b2_r, nbr_ref, h2_ref, h2s_ref = rest

        i = pl.program_id(0)
        col0 = ws_ref[i] * CG
        pos_q, x_q, bat_q = pos_q_r[...], x_q_r[...], bat_q_r[...]
        posT = jnp.concatenate([r[...] for r in posT_rs], axis=1)
        batT = jnp.concatenate([r[...] for r in batT_rs], axis=1)
        pos_f = jnp.concatenate([r[...] for r in posf_rs], axis=0)
        W1x, W1p, b1 = W1x_r[...], W1p_r[...], b1_r[...]
        W2, b2 = W2_r[...], b2_r[...]

        ii = jax.lax.broadcasted_iota(jnp.int32, (BQ, width), 1)
        d0 = pos_q[:, 0:1] - posT[0:1, :]
        d1 = pos_q[:, 1:2] - posT[1:2, :]
        d2c = pos_q[:, 2:3] - posT[2:3, :]
        d = (d0 * d0 + d1 * d1) + d2c * d2c
        d = jnp.where((bat_q != batT) | (ii + col0 >= N), jnp.inf, d)

        base = jnp.dot(x_q, W1x, preferred_element_type=jnp.float32) + b1
        h1s = jax.nn.relu(base)
        h2s_ref[...] = jax.nn.relu(
            jnp.dot(h1s, W2, preferred_element_type=jnp.float32) + b2)

        for k in range(K):
            m = jnp.min(d, axis=1, keepdims=True)
            masked = jnp.where(d == m, ii, jnp.int32(width))
            amin = jnp.min(masked, axis=1)
            onehot = ii == amin[:, None]
            nbr_ref[k, :] = amin + col0
            pos_n = jnp.dot(onehot.astype(jnp.float32), pos_f,
                            preferred_element_type=jnp.float32)
            d = jnp.where(onehot, jnp.inf, d)
            pd = pos_q - pos_n
            h1 = jax.nn.relu(
                base + jnp.dot(pd, W1p, preferred_element_type=jnp.float32))
            h2_ref[k] = jax.nn.relu(
                jnp.dot(h1, W2, preferred_element_type=jnp.float32) + b2)

    return body


def _chunk_map(c):
    return lambda i, ws: (0, jnp.minimum(ws[i] + c, NCH - 1))


def _chunk_map_r(c):
    return lambda i, ws: (jnp.minimum(ws[i] + c, NCH - 1), 0)


def _knn_call(nch, ws, pos, x, bat_col, posT, bat_row, W1x, W1p, b1, W2, b2):
    in_specs = [
        pl.BlockSpec((BQ, 3), lambda i, ws: (i, 0)),
        pl.BlockSpec((BQ, 128), lambda i, ws: (i, 0)),
        pl.BlockSpec((BQ, 1), lambda i, ws: (i, 0)),
    ]
    in_specs += [pl.BlockSpec((3, CG), _chunk_map(c)) for c in range(nch)]
    in_specs += [pl.BlockSpec((1, CG), _chunk_map(c)) for c in range(nch)]
    in_specs += [pl.BlockSpec((CG, 3), _chunk_map_r(c)) for c in range(nch)]
    in_specs += [
        pl.BlockSpec((128, 128), lambda i, ws: (0, 0)),
        pl.BlockSpec((3, 128), lambda i, ws: (0, 0)),
        pl.BlockSpec((1, 128), lambda i, ws: (0, 0)),
        pl.BlockSpec((128, 256), lambda i, ws: (0, 0)),
        pl.BlockSpec((1, 256), lambda i, ws: (0, 0)),
    ]
    args = ([pos, x, bat_col] + [posT] * nch + [bat_row] * nch + [pos] * nch
            + [W1x, W1p, b1, W2, b2])
    return pl.pallas_call(
        _make_knn_body(nch),
        grid_spec=pltpu.PrefetchScalarGridSpec(
            num_scalar_prefetch=1,
            grid=(NBLK,),
            in_specs=in_specs,
            out_specs=[
                pl.BlockSpec((K, BQ), lambda i, ws: (0, i)),
                pl.BlockSpec((K, BQ, 256), lambda i, ws: (0, i, 0)),
                pl.BlockSpec((BQ, 256), lambda i, ws: (i, 0)),
            ],
        ),
        out_shape=[
            jax.ShapeDtypeStruct((K, N), jnp.int32),
            jax.ShapeDtypeStruct((K, N, 256), jnp.float32),
            jax.ShapeDtypeStruct((N, 256), jnp.float32),
        ],
    )(ws, *args)


def _scatter_kernel(nbr_sm, h2_ref, h2s_ref, out_ref):
    i = pl.program_id(0)

    @pl.when(i == 0)
    def _init():
        out_ref[...] = h2s_ref[...]

    def body(q, carry):
        for k in range(K):
            dst = nbr_sm[k, i * BQ + q]
            row = h2_ref[k, pl.ds(q, 1), :]
            cur = out_ref[pl.ds(dst, 1), :]
            out_ref[pl.ds(dst, 1), :] = jnp.maximum(cur, row)
        return carry

    jax.lax.fori_loop(0, BQ, body, 0)


def _final_kernel(agg, Wg, bg, out_ref):
    out_ref[...] = jax.nn.relu(
        jnp.dot(agg[...], Wg[...], preferred_element_type=jnp.float32) + bg[...])


def kernel(x, pos, batch, W1, b1, W2, b2, Wg, bg):
    bat32 = batch.astype(jnp.int32)
    batf = bat32.astype(jnp.float32)
    posT = pos.T
    W1x = W1[:128, :]
    W1p = W1[128:, :]
    b1r = b1.reshape(1, 128)
    b2r = b2.reshape(1, 256)
    bat_col = batf.reshape(N, 1)
    bat_row = batf.reshape(1, N)

    counts = jnp.bincount(bat32, length=8)
    off = jnp.cumsum(counts) - counts
    ends = off + counts
    qb = jnp.arange(NBLK)
    b_first = bat32[qb * BQ]
    b_last = bat32[qb * BQ + BQ - 1]
    ws = (off[b_first] // CG).astype(jnp.int32)
    fits = jnp.all(ends[b_last] - ws * CG <= 6 * CG)
    ws_full = jnp.zeros((NBLK,), jnp.int32)

    common = (pos, x, bat_col, posT, bat_row, W1x, W1p, b1r, W2, b2r)
    nbr, h2, h2s = jax.lax.cond(
        fits,
        lambda ws_, ws_full_, *a: _knn_call(6, ws_, *a),
        lambda ws_, ws_full_, *a: _knn_call(NCH, ws_full_, *a),
        ws, ws_full, *common)

    agg = pl.pallas_call(
        _scatter_kernel,
        grid_spec=pltpu.PrefetchScalarGridSpec(
            num_scalar_prefetch=1,
            grid=(NBLK,),
            in_specs=[
                pl.BlockSpec((K, BQ, 256), lambda i, s: (0, i, 0)),
                pl.BlockSpec((N, 256), lambda i, s: (0, 0)),
            ],
            out_specs=pl.BlockSpec((N, 256), lambda i, s: (0, 0)),
        ),
        out_shape=jax.ShapeDtypeStruct((N, 256), jnp.float32),
    )(nbr, h2, h2s)

    out = pl.pallas_call(
        _final_kernel,
        grid=(NBLK,),
        in_specs=[
            pl.BlockSpec((BQ, 256), lambda i: (i, 0)),
            pl.BlockSpec((256, 256), lambda i: (0, 0)),
            pl.BlockSpec((1, 256), lambda i: (0, 0)),
        ],
        out_specs=pl.BlockSpec((BQ, 256), lambda i: (i, 0)),
        out_shape=jax.ShapeDtypeStruct((N, 256), jnp.float32),
    )(agg, Wg, bg.reshape(1, 256))

    return (out, pos, batch)
